# baseline (device time: 93512 ns/iter reference)
import jax
import jax.numpy as jnp
from jax import lax
from jax.experimental import pallas as pl
from jax.experimental.pallas import tpu as pltpu

_EPS = 1e-5
_BM = 512


def kernel(x, dy, gamma):
    del gamma
    m, d = x.shape
    nb = m // _BM

    def body(x_ref, dy_ref, out_ref, acc_ref, recv_ref, send_sem, recv_sem):
        i = pl.program_id(0)

        xv = x_ref[:, :]
        dyv = dy_ref[:, :]

        bm = xv.shape[0]
        dd = xv.shape[1]
        ones_d = jnp.ones((dd, 1), jnp.float32)
        hi = lax.Precision.HIGHEST
        row_x = lax.dot_general(
            xv, ones_d, (((1,), (0,)), ((), ())), precision=hi)
        row_xx = lax.dot_general(
            xv * xv, ones_d, (((1,), (0,)), ((), ())), precision=hi)
        mu = row_x / dd
        var = row_xx / dd - mu * mu
        a = lax.rsqrt(var + _EPS)
        b = mu * a

        p = xv * dyv
        s1 = lax.dot_general(
            a, p, (((0,), (0,)), ((), ())), precision=hi)
        w2 = jnp.concatenate([jnp.ones((bm, 1), jnp.float32), b], axis=1)
        s2 = lax.dot_general(
            w2, dyv, (((0,), (0,)), ((), ())), precision=hi)
        dgamma = s1 - s2[1:2, :]
        part = jnp.concatenate([dgamma, s2[0:1, :]], axis=0)

        @pl.when(i == 0)
        def _():
            acc_ref[:, :] = part

        @pl.when(i > 0)
        def _():
            acc_ref[:, :] = acc_ref[:, :] + part

        @pl.when(i == nb - 1)
        def _():
            my_x = lax.axis_index("x")
            my_y = lax.axis_index("y")
            peer = (my_x, 1 - my_y)

            barrier = pltpu.get_barrier_semaphore()
            pl.semaphore_signal(
                barrier, inc=1,
                device_id=peer, device_id_type=pl.DeviceIdType.MESH,
            )
            pl.semaphore_wait(barrier, 1)

            rdma = pltpu.make_async_remote_copy(
                src_ref=acc_ref,
                dst_ref=recv_ref,
                send_sem=send_sem,
                recv_sem=recv_sem,
                device_id=peer,
                device_id_type=pl.DeviceIdType.MESH,
            )
            rdma.start()
            rdma.wait()

            out_ref[:, :] = acc_ref[:, :] + recv_ref[:, :]

    return pl.pallas_call(
        body,
        grid=(nb,),
        in_specs=[
            pl.BlockSpec((_BM, d), lambda i: (i, 0)),
            pl.BlockSpec((_BM, d), lambda i: (i, 0)),
        ],
        out_specs=pl.BlockSpec((2, d), lambda i: (0, 0)),
        out_shape=jax.ShapeDtypeStruct((2, d), jnp.float32),
        scratch_shapes=[
            pltpu.VMEM((2, d), jnp.float32),
            pltpu.VMEM((2, d), jnp.float32),
            pltpu.SemaphoreType.DMA,
            pltpu.SemaphoreType.DMA,
        ],
        compiler_params=pltpu.CompilerParams(
            collective_id=0,
            dimension_semantics=("arbitrary",),
        ),
    )(x, dy)


# device time: 19194 ns/iter; 4.8719x vs baseline; 4.8719x over previous
import jax
import jax.numpy as jnp
from jax import lax
from jax.experimental import pallas as pl
from jax.experimental.pallas import tpu as pltpu

_EPS = 1e-5
_BM = 512


def kernel(x, dy, gamma):
    del gamma
    m, d = x.shape
    half = m // 2
    nb = half // _BM

    def body(off_ref, x_ref, dy_ref, out_ref,
             acc_ref, recv_x_ref, recv_y_ref, send_sems, recv_sems):
        i = pl.program_id(0)

        xv = x_ref[:, :]
        dyv = dy_ref[:, :]
        mu = jnp.mean(xv, axis=1, keepdims=True)
        xc = xv - mu
        var = jnp.mean(xc * xc, axis=1, keepdims=True)
        xhat = xc * lax.rsqrt(var + _EPS)
        dgamma = jnp.sum(dyv * xhat, axis=0, keepdims=True)
        dbeta = jnp.sum(dyv, axis=0, keepdims=True)
        part = jnp.concatenate([dgamma, dbeta], axis=0)

        @pl.when(i == 0)
        def _():
            acc_ref[:, :] = part

        @pl.when(i > 0)
        def _():
            acc_ref[:, :] = acc_ref[:, :] + part

        @pl.when(i == nb - 1)
        def _():
            my_x = lax.axis_index("x")
            my_y = lax.axis_index("y")
            x_peer = (1 - my_x, my_y)
            y_peer = (my_x, 1 - my_y)

            barrier = pltpu.get_barrier_semaphore()
            for peer in (x_peer, y_peer):
                pl.semaphore_signal(
                    barrier, inc=1,
                    device_id=peer, device_id_type=pl.DeviceIdType.MESH,
                )
            pl.semaphore_wait(barrier, 2)

            rdma1 = pltpu.make_async_remote_copy(
                src_ref=acc_ref,
                dst_ref=recv_x_ref,
                send_sem=send_sems.at[0],
                recv_sem=recv_sems.at[0],
                device_id=x_peer,
                device_id_type=pl.DeviceIdType.MESH,
            )
            rdma1.start()
            rdma1.wait()
            acc_ref[:, :] = acc_ref[:, :] + recv_x_ref[:, :]

            rdma2 = pltpu.make_async_remote_copy(
                src_ref=acc_ref,
                dst_ref=recv_y_ref,
                send_sem=send_sems.at[1],
                recv_sem=recv_sems.at[1],
                device_id=y_peer,
                device_id_type=pl.DeviceIdType.MESH,
            )
            rdma2.start()
            rdma2.wait()

            out_ref[:, :] = acc_ref[:, :] + recv_y_ref[:, :]

    grid_spec = pltpu.PrefetchScalarGridSpec(
        num_scalar_prefetch=1,
        grid=(nb,),
        in_specs=[
            pl.BlockSpec((_BM, d), lambda i, off: (off[0] + i, 0)),
            pl.BlockSpec((_BM, d), lambda i, off: (off[0] + i, 0)),
        ],
        out_specs=pl.BlockSpec((2, d), lambda i, off: (0, 0)),
        scratch_shapes=[
            pltpu.VMEM((2, d), jnp.float32),
            pltpu.VMEM((2, d), jnp.float32),
            pltpu.VMEM((2, d), jnp.float32),
            pltpu.SemaphoreType.DMA((2,)),
            pltpu.SemaphoreType.DMA((2,)),
        ],
    )

    block_off = (lax.axis_index("x") * nb).astype(jnp.int32).reshape((1,))

    return pl.pallas_call(
        body,
        grid_spec=grid_spec,
        out_shape=jax.ShapeDtypeStruct((2, d), jnp.float32),
        compiler_params=pltpu.CompilerParams(
            collective_id=0,
            dimension_semantics=("arbitrary",),
        ),
    )(block_off, x, dy)


# device time: 18139 ns/iter; 5.1553x vs baseline; 1.0582x over previous
import jax
import jax.numpy as jnp
from jax import lax
from jax.experimental import pallas as pl
from jax.experimental.pallas import tpu as pltpu

_EPS = 1e-5
_BM = 512


def kernel(x, dy, gamma):
    del gamma
    m, d = x.shape
    half = m // 2
    nb = half // _BM

    def body(off_ref, x_ref, dy_ref, out_ref,
             acc_ref, last_ref, recv_early, recv_last, send_sems, recv_sems):
        i = pl.program_id(0)

        xv = x_ref[:, :]
        dyv = dy_ref[:, :]
        mu = jnp.mean(xv, axis=1, keepdims=True)
        xc = xv - mu
        var = jnp.mean(xc * xc, axis=1, keepdims=True)
        xhat = xc * lax.rsqrt(var + _EPS)
        dgamma = jnp.sum(dyv * xhat, axis=0, keepdims=True)
        dbeta = jnp.sum(dyv, axis=0, keepdims=True)
        part = jnp.concatenate([dgamma, dbeta], axis=0)

        my_x = lax.axis_index("x")
        my_y = lax.axis_index("y")
        peers = [
            (1 - my_x, my_y),
            (my_x, 1 - my_y),
            (1 - my_x, 1 - my_y),
        ]

        def mk(phase, k, src, dst):
            return pltpu.make_async_remote_copy(
                src_ref=src,
                dst_ref=dst,
                send_sem=send_sems.at[phase, k],
                recv_sem=recv_sems.at[phase, k],
                device_id=peers[k],
                device_id_type=pl.DeviceIdType.MESH,
            )

        @pl.when(i == 0)
        def _():
            barrier = pltpu.get_barrier_semaphore()
            for p in peers:
                pl.semaphore_signal(
                    barrier, inc=1,
                    device_id=p, device_id_type=pl.DeviceIdType.MESH,
                )
            pl.semaphore_wait(barrier, 3)
            acc_ref[:, :] = part

        @pl.when(jnp.logical_and(i > 0, i <= nb - 2))
        def _():
            acc_ref[:, :] = acc_ref[:, :] + part

        @pl.when(i == nb - 2)
        def _():
            for k in range(3):
                mk(0, k, acc_ref, recv_early.at[k]).start()

        @pl.when(i == nb - 1)
        def _():
            last_ref[:, :] = part
            for k in range(3):
                mk(1, k, last_ref, recv_last.at[k]).start()
            total = acc_ref[:, :] + part
            for k in range(3):
                mk(0, k, acc_ref, recv_early.at[k]).wait_recv()
                mk(1, k, last_ref, recv_last.at[k]).wait_recv()
                total = total + recv_early[k, :, :] + recv_last[k, :, :]
            out_ref[:, :] = total
            for k in range(3):
                mk(0, k, acc_ref, recv_early.at[k]).wait_send()
                mk(1, k, last_ref, recv_last.at[k]).wait_send()

    grid_spec = pltpu.PrefetchScalarGridSpec(
        num_scalar_prefetch=1,
        grid=(nb,),
        in_specs=[
            pl.BlockSpec((_BM, d), lambda i, off: (off[0] + i, 0)),
            pl.BlockSpec((_BM, d), lambda i, off: (off[0] + i, 0)),
        ],
        out_specs=pl.BlockSpec((2, d), lambda i, off: (0, 0)),
        scratch_shapes=[
            pltpu.VMEM((2, d), jnp.float32),
            pltpu.VMEM((2, d), jnp.float32),
            pltpu.VMEM((3, 2, d), jnp.float32),
            pltpu.VMEM((3, 2, d), jnp.float32),
            pltpu.SemaphoreType.DMA((2, 3)),
            pltpu.SemaphoreType.DMA((2, 3)),
        ],
    )

    block_off = (lax.axis_index("x") * nb).astype(jnp.int32).reshape((1,))

    return pl.pallas_call(
        body,
        grid_spec=grid_spec,
        out_shape=jax.ShapeDtypeStruct((2, d), jnp.float32),
        compiler_params=pltpu.CompilerParams(
            collective_id=0,
            dimension_semantics=("arbitrary",),
        ),
    )(block_off, x, dy)
